# baseline (speedup 1.0000x reference)
import functools

import jax
import jax.numpy as jnp
from jax import lax
from jax.experimental import pallas as pl
from jax.experimental.pallas import tpu as pltpu
from jax.experimental.pallas import tpu_sc as plsc

_LANES = 16
_NC = 2
_NS = 16
_NW = _NC * _NS
_UNROLL = 4
_NCH = 2


@functools.lru_cache(maxsize=None)
def _build_sc_loss(n: int, e: int, chunk: int):
    nv = chunk // _LANES
    assert chunk % (_NCH * _UNROLL * _LANES) == 0 and nv % _UNROLL == 0
    lbase_max = (n - chunk) // 8 * 8
    mesh = plsc.VectorSubcoreMesh(core_axis_name="c", subcore_axis_name="s")

    @functools.partial(
        pl.kernel,
        out_type=jax.ShapeDtypeStruct((_NC * _LANES,), jnp.float32),
        mesh=mesh,
        compiler_params=pltpu.CompilerParams(needs_layout_passes=False),
        scratch_types=[
            pltpu.VMEM((n,), jnp.int32),
            *[pltpu.VMEM((chunk // _NCH,), jnp.int32)
              for _ in range(4 * _NCH)],
            pltpu.VMEM((_LANES,), jnp.float32),
            pltpu.VMEM((_NS * _LANES,), jnp.float32),
            pltpu.VMEM_SHARED((_NS * _LANES,), jnp.float32),
            pltpu.SemaphoreType.DMA,
            pltpu.SemaphoreType.DMA,
            pltpu.SemaphoreType.DMA,
        ],
    )
    def sc_loss(ids_hbm, eflat_hbm, pack_hbm, out_hbm, table_v,
                *rest):
        bufs, (acc_v, red_v, shared, sem_t, sem_l, sem_g) = (
            rest[:4 * _NCH], rest[4 * _NCH:])
        ids_l = bufs[0::4]
        ids2_l = bufs[1::4]
        src_l = bufs[2::4]
        dst_l = bufs[3::4]
        c = lax.axis_index("c")
        s = lax.axis_index("s")
        wid = s * _NC + c
        owned_lo = wid * chunk
        lbase = jnp.minimum(owned_lo, lbase_max)

        cp_tab = pltpu.async_copy(pack_hbm, table_v, sem_t)
        csz = chunk // _NCH
        cp_ids = [pltpu.async_copy(ids_hbm.at[pl.ds(lbase + ch * csz, csz)],
                                   ids_l[ch], sem_l)
                  for ch in range(_NCH)]

        cvec = csz // _LANES
        iota = lax.iota(jnp.int32, 16)
        gathers = []
        for ch in range(_NCH):
            cp_ids[ch].wait()
            idsr = ids_l[ch]
            ids2r = ids2_l[ch]

            def prep_body(k, carry, idsr=idsr, ids2r=ids2r):
                o = k * (_LANES * _UNROLL)
                for u in range(_UNROLL):
                    ou = o + u * _LANES
                    v = idsr[pl.ds(ou, _LANES)]
                    b = v + (v & -128)
                    idsr[pl.ds(ou, _LANES)] = b
                    ids2r[pl.ds(ou, _LANES)] = b + 128
                return carry
            lax.fori_loop(0, cvec // _UNROLL, prep_body, 0)
            gathers.append(pltpu.async_copy(
                eflat_hbm.at[idsr], src_l[ch], sem_g))
            gathers.append(pltpu.async_copy(
                eflat_hbm.at[ids2r], dst_l[ch], sem_g))

        cp_tab.wait()

        def zero_body(k, acc):
            o = k * (_LANES * _UNROLL)
            for u in range(_UNROLL):
                ou = o + u * _LANES
                w = table_v[pl.ds(lbase + ou, _LANES)]
                p = lax.bitcast_convert_type(w & -8, jnp.float32)
                glob = lbase + ou + iota
                m = ((w & 7) == 0) & (glob >= owned_lo) & (glob < n)
                acc = acc + jnp.where(m, p * p, 0.0)
            return acc
        acc = lax.fori_loop(0, nv // _UNROLL, zero_body,
                            jnp.zeros((_LANES,), jnp.float32))

        def make_edge_body(ch):
            srcr = src_l[ch]
            dstr = dst_l[ch]
            cb = ch * csz

            def edge_body(k, acc):
                o = k * (_LANES * _UNROLL)
                for u in range(_UNROLL):
                    ou = o + u * _LANES
                    si = srcr[pl.ds(ou, _LANES)]
                    di = dstr[pl.ds(ou, _LANES)]
                    wi = plsc.load_gather(table_v, [si])
                    wj = plsc.load_gather(table_v, [di])
                    li = wi & 7
                    lj = wj & 7
                    pi = lax.bitcast_convert_type(wi & -8, jnp.float32)
                    pj = lax.bitcast_convert_type(wj & -8, jnp.float32)
                    dp = pi - pj
                    margin = jnp.abs(li - lj).astype(jnp.float32)
                    h = jnp.maximum(margin - jnp.abs(dp), 0.0)
                    contrib = jnp.where(li == lj, dp * dp, 10.0 * h * h)
                    glob = lbase + cb + ou + iota
                    m = (glob >= owned_lo) & (glob < n)
                    acc = acc + jnp.where(m, contrib, 0.0)
                return acc
            return edge_body

        for ch in range(_NCH):
            gathers[2 * ch].wait()
            gathers[2 * ch + 1].wait()
            acc = lax.fori_loop(0, cvec // _UNROLL, make_edge_body(ch), acc)

        acc_v[...] = acc * (1.0 / n)

        pltpu.sync_copy(acc_v, shared.at[pl.ds(s * _LANES, _LANES)])
        plsc.subcore_barrier()

        @pl.when(s == 0)
        def _():
            pltpu.sync_copy(shared, red_v)

            def red_body(i, tot):
                return tot + red_v[pl.ds(i * _LANES, _LANES)]
            tot = lax.fori_loop(0, _NS, red_body,
                                jnp.zeros((_LANES,), jnp.float32))
            acc_v[...] = tot
            pltpu.sync_copy(acc_v,
                            out_hbm.at[pl.ds(c * _LANES, _LANES)])

    return sc_loss


def kernel(y_true, y_pred, src, dst, edge_index, edge_ids):
    n = y_true.shape[0]
    e = edge_index.shape[1]
    g = _NW * _NCH * _UNROLL * _LANES
    npad = -(-n // g) * g
    chunk = npad // _NW
    pack = ((jax.lax.bitcast_convert_type(y_pred.astype(jnp.float32),
                                          jnp.int32) & -8)
            | y_true.astype(jnp.int32))
    eflat = (edge_index.astype(jnp.int32)
             .reshape(2, e // 128, 128)
             .transpose(1, 0, 2)
             .reshape(-1))
    part = _build_sc_loss(n, e, chunk)(edge_ids.astype(jnp.int32), eflat,
                                       pack)
    return jnp.sum(part)


# speedup vs baseline: 1.0100x; 1.0100x over previous
import functools

import jax
import jax.numpy as jnp
from jax import lax
from jax.experimental import pallas as pl
from jax.experimental.pallas import tpu as pltpu
from jax.experimental.pallas import tpu_sc as plsc

_LANES = 16
_NC = 2
_NS = 16
_NW = _NC * _NS
_UNROLL = 2
_NCH = 2


@functools.lru_cache(maxsize=None)
def _build_sc_loss(n: int, e: int, chunk: int):
    nv = chunk // _LANES
    assert chunk % (_NCH * _UNROLL * _LANES) == 0 and nv % _UNROLL == 0
    lbase_max = (n - chunk) // 8 * 8
    mesh = plsc.VectorSubcoreMesh(core_axis_name="c", subcore_axis_name="s")

    @functools.partial(
        pl.kernel,
        out_type=jax.ShapeDtypeStruct((_NC * _LANES,), jnp.float32),
        mesh=mesh,
        compiler_params=pltpu.CompilerParams(needs_layout_passes=False),
        scratch_types=[
            pltpu.VMEM((n,), jnp.int32),
            *[pltpu.VMEM((chunk // _NCH,), jnp.int32)
              for _ in range(4 * _NCH)],
            pltpu.VMEM((_LANES,), jnp.float32),
            pltpu.VMEM((_NS * _LANES,), jnp.float32),
            pltpu.VMEM_SHARED((_NS * _LANES,), jnp.float32),
            pltpu.SemaphoreType.DMA,
            pltpu.SemaphoreType.DMA,
            pltpu.SemaphoreType.DMA,
        ],
    )
    def sc_loss(ids_hbm, eflat_hbm, pack_hbm, out_hbm, table_v,
                *rest):
        bufs, (acc_v, red_v, shared, sem_t, sem_l, sem_g) = (
            rest[:4 * _NCH], rest[4 * _NCH:])
        ids_l = bufs[0::4]
        ids2_l = bufs[1::4]
        src_l = bufs[2::4]
        dst_l = bufs[3::4]
        c = lax.axis_index("c")
        s = lax.axis_index("s")
        wid = s * _NC + c
        owned_lo = wid * chunk
        lbase = jnp.minimum(owned_lo, lbase_max)

        cp_tab = pltpu.async_copy(pack_hbm, table_v, sem_t)
        csz = chunk // _NCH
        cp_ids = [pltpu.async_copy(ids_hbm.at[pl.ds(lbase + ch * csz, csz)],
                                   ids_l[ch], sem_l)
                  for ch in range(_NCH)]

        cvec = csz // _LANES
        iota = lax.iota(jnp.int32, 16)
        gathers = []
        for ch in range(_NCH):
            cp_ids[ch].wait()
            idsr = ids_l[ch]
            ids2r = ids2_l[ch]

            def prep_body(k, carry, idsr=idsr, ids2r=ids2r):
                o = k * (_LANES * _UNROLL)
                for u in range(_UNROLL):
                    ou = o + u * _LANES
                    v = idsr[pl.ds(ou, _LANES)]
                    b = v + (v & -128)
                    idsr[pl.ds(ou, _LANES)] = b
                    ids2r[pl.ds(ou, _LANES)] = b + 128
                return carry
            lax.fori_loop(0, cvec // _UNROLL, prep_body, 0)
            gathers.append(pltpu.async_copy(
                eflat_hbm.at[idsr], src_l[ch], sem_g))
            gathers.append(pltpu.async_copy(
                eflat_hbm.at[ids2r], dst_l[ch], sem_g))

        cp_tab.wait()

        def zero_body(k, acc):
            o = k * (_LANES * _UNROLL)
            for u in range(_UNROLL):
                ou = o + u * _LANES
                w = table_v[pl.ds(lbase + ou, _LANES)]
                p = lax.bitcast_convert_type(w & -8, jnp.float32)
                glob = lbase + ou + iota
                m = ((w & 7) == 0) & (glob >= owned_lo) & (glob < n)
                acc = acc + jnp.where(m, p * p, 0.0)
            return acc
        acc = lax.fori_loop(0, nv // _UNROLL, zero_body,
                            jnp.zeros((_LANES,), jnp.float32))

        def make_edge_body(ch):
            srcr = src_l[ch]
            dstr = dst_l[ch]
            cb = ch * csz

            def edge_body(k, acc):
                o = k * (_LANES * _UNROLL)
                for u in range(_UNROLL):
                    ou = o + u * _LANES
                    si = srcr[pl.ds(ou, _LANES)]
                    di = dstr[pl.ds(ou, _LANES)]
                    wi = plsc.load_gather(table_v, [si])
                    wj = plsc.load_gather(table_v, [di])
                    li = wi & 7
                    lj = wj & 7
                    pi = lax.bitcast_convert_type(wi & -8, jnp.float32)
                    pj = lax.bitcast_convert_type(wj & -8, jnp.float32)
                    dp = pi - pj
                    margin = jnp.abs(li - lj).astype(jnp.float32)
                    h = jnp.maximum(margin - jnp.abs(dp), 0.0)
                    contrib = jnp.where(li == lj, dp * dp, 10.0 * h * h)
                    glob = lbase + cb + ou + iota
                    m = (glob >= owned_lo) & (glob < n)
                    acc = acc + jnp.where(m, contrib, 0.0)
                return acc
            return edge_body

        for ch in range(_NCH):
            gathers[2 * ch].wait()
            gathers[2 * ch + 1].wait()
            acc = lax.fori_loop(0, cvec // _UNROLL, make_edge_body(ch), acc)

        acc_v[...] = acc * (1.0 / n)

        pltpu.sync_copy(acc_v, shared.at[pl.ds(s * _LANES, _LANES)])
        plsc.subcore_barrier()

        @pl.when(s == 0)
        def _():
            pltpu.sync_copy(shared, red_v)

            def red_body(i, tot):
                return tot + red_v[pl.ds(i * _LANES, _LANES)]
            tot = lax.fori_loop(0, _NS, red_body,
                                jnp.zeros((_LANES,), jnp.float32))
            acc_v[...] = tot
            pltpu.sync_copy(acc_v,
                            out_hbm.at[pl.ds(c * _LANES, _LANES)])

    return sc_loss


def kernel(y_true, y_pred, src, dst, edge_index, edge_ids):
    n = y_true.shape[0]
    e = edge_index.shape[1]
    g = _NW * _NCH * _UNROLL * _LANES
    npad = -(-n // g) * g
    chunk = npad // _NW
    pack = ((jax.lax.bitcast_convert_type(y_pred.astype(jnp.float32),
                                          jnp.int32) & -8)
            | y_true.astype(jnp.int32))
    eflat = (edge_index.astype(jnp.int32)
             .reshape(2, e // 128, 128)
             .transpose(1, 0, 2)
             .reshape(-1))
    part = _build_sc_loss(n, e, chunk)(edge_ids.astype(jnp.int32), eflat,
                                       pack)
    return jnp.sum(part)
